# Initial kernel scaffold; baseline (speedup 1.0000x reference)
import functools

import jax
import jax.numpy as jnp
from jax import lax
from jax.experimental import pallas as pl
from jax.experimental.pallas import tpu as pltpu
from jax.experimental.pallas import tpu_sc as plsc

_L = 128
_G = 16
_CHUNK = _G * _L


def _fold_table(wte, W, b):
    def body(wte_ref, w_ref, b_ref, out_ref):
        out_ref[...] = lax.dot_general(
            wte_ref[...], w_ref[...],
            (((1,), (1,)), ((), ())),
            preferred_element_type=jnp.float32) + b_ref[...]

    return pl.pallas_call(
        body,
        out_shape=jax.ShapeDtypeStruct(wte.shape, jnp.float32),
    )(wte, W, b.reshape(1, -1))


def _sc_gather(idx2d, table):
    ng = idx2d.shape[0]
    d = table.shape[1]
    info = plsc.get_sparse_core_info()
    nw = info.num_cores * info.num_subcores
    gpw = ng // nw
    nchunks = gpw // _G
    n = ng * _L

    mesh = plsc.VectorSubcoreMesh(core_axis_name="c", subcore_axis_name="s")

    @functools.partial(
        pl.kernel,
        out_type=jax.ShapeDtypeStruct((n, d), jnp.float32),
        mesh=mesh,
        scratch_types=[
            pltpu.VMEM((_G, _L), jnp.int32),
            pltpu.VMEM((_CHUNK, d), jnp.float32),
            pltpu.SemaphoreType.DMA,
        ],
    )
    def k(idx_hbm, table_hbm, out_hbm, idx_v, rows_v, sem):
        wid = lax.axis_index("s") * info.num_cores + lax.axis_index("c")
        g_base = wid * gpw

        def chunk(c, carry):
            g0 = g_base + c * _G
            pltpu.sync_copy(idx_hbm.at[pl.ds(g0, _G)], idx_v)
            copies = [
                pltpu.async_copy(table_hbm.at[idx_v.at[j]],
                                 rows_v.at[pl.ds(j * _L, _L)], sem)
                for j in range(_G)
            ]
            for cp in copies:
                cp.wait()
            pltpu.sync_copy(rows_v, out_hbm.at[pl.ds(g0 * _L, _CHUNK)])
            return carry

        lax.fori_loop(0, nchunks, chunk, 0)

    return k(idx2d, table)


def kernel(x, wte, W, b):
    bsz, seq = x.shape
    table2 = _fold_table(wte, W, b)
    idx2d = x.reshape(-1, _L).astype(jnp.int32)
    out = _sc_gather(idx2d, table2)
    return out.reshape(bsz, seq, wte.shape[1])


# speedup vs baseline: 5.8241x; 5.8241x over previous
import functools

import jax
import jax.numpy as jnp
from jax import lax
from jax.experimental import pallas as pl
from jax.experimental.pallas import tpu as pltpu
from jax.experimental.pallas import tpu_sc as plsc

_L = 128
_G = 16
_CHUNK = _G * _L


def _fold_table(wte, W, b):
    def body(wte_ref, w_ref, b_ref, out_ref):
        out_ref[...] = lax.dot_general(
            wte_ref[...], w_ref[...],
            (((1,), (1,)), ((), ())),
            preferred_element_type=jnp.float32) + b_ref[...]

    return pl.pallas_call(
        body,
        out_shape=jax.ShapeDtypeStruct(wte.shape, jnp.float32),
    )(wte, W, b.reshape(1, -1))


def _sc_gather(idx2d, table):
    ng = idx2d.shape[0]
    d = table.shape[1]
    info = plsc.get_sparse_core_info()
    nw = info.num_cores * info.num_subcores
    gpw = ng // nw
    nchunks = gpw // _G
    n = ng * _L

    mesh = plsc.VectorSubcoreMesh(core_axis_name="c", subcore_axis_name="s")

    @functools.partial(
        pl.kernel,
        out_type=jax.ShapeDtypeStruct((n, d), jnp.float32),
        mesh=mesh,
        scratch_types=[
            pltpu.VMEM((_G, _L), jnp.int32),
            pltpu.VMEM((_CHUNK, d), jnp.float32),
            pltpu.SemaphoreType.DMA,
        ],
        compiler_params=pltpu.CompilerParams(use_tc_tiling_on_sc=False),
    )
    def k(idx_hbm, table_hbm, out_hbm, idx_v, rows_v, sem):
        wid = lax.axis_index("s") * info.num_cores + lax.axis_index("c")
        g_base = wid * gpw

        def chunk(c, carry):
            g0 = g_base + c * _G
            pltpu.sync_copy(idx_hbm.at[pl.ds(g0, _G)], idx_v)
            copies = [
                pltpu.async_copy(table_hbm.at[idx_v.at[j]],
                                 rows_v.at[pl.ds(j * _L, _L)], sem)
                for j in range(_G)
            ]
            for cp in copies:
                cp.wait()
            pltpu.sync_copy(rows_v, out_hbm.at[pl.ds(g0 * _L, _CHUNK)])
            return carry

        lax.fori_loop(0, nchunks, chunk, 0)

    return k(idx2d, table)


def kernel(x, wte, W, b):
    bsz, seq = x.shape
    table2 = _fold_table(wte, W, b)
    idx2d = x.reshape(-1, _L).astype(jnp.int32)
    out = _sc_gather(idx2d, table2)
    return out.reshape(bsz, seq, wte.shape[1])
